# baseline (speedup 1.0000x reference)
import jax
from jax.experimental import pallas as pl
from jax.experimental.pallas import tpu as pltpu


def _copy_body(x_ref, o_ref):
    o_ref[...] = x_ref[...]


def kernel(x):
    B, T, C = x.shape
    x2 = x.reshape(B * T, C)
    rows = B * T
    block_rows = 4096
    out = pl.pallas_call(
        _copy_body,
        out_shape=jax.ShapeDtypeStruct((rows, C), x.dtype),
        grid=(rows // block_rows,),
        in_specs=[pl.BlockSpec((block_rows, C), lambda i: (i, 0))],
        out_specs=pl.BlockSpec((block_rows, C), lambda i: (i, 0)),
        compiler_params=pltpu.CompilerParams(
            dimension_semantics=("arbitrary",),
        ),
    )(x2)
    return out.reshape(B, T, C)


# speedup vs baseline: 1.0001x; 1.0001x over previous
import jax
import jax.numpy as jnp
from jax.experimental import pallas as pl
from jax.experimental.pallas import tpu as pltpu

_CHUNK = 4096
_K = 4
_L = 2


def _body(x_ref, o_ref, buf, in_sems, out_sems):
    rows = x_ref.shape[0]
    n = rows // _CHUNK

    def in_copy(i):
        s = i % _K
        return pltpu.make_async_copy(
            x_ref.at[pl.ds(i * _CHUNK, _CHUNK)], buf.at[s], in_sems.at[s]
        )

    def out_copy(i):
        s = i % _K
        return pltpu.make_async_copy(
            buf.at[s], o_ref.at[pl.ds(i * _CHUNK, _CHUNK)], out_sems.at[s]
        )

    waited = set()
    for j in range(min(_L, n)):
        in_copy(j).start()
    for i in range(n):
        in_copy(i).wait()
        out_copy(i).start()
        nxt = i + _L
        if nxt < n:
            if nxt >= _K:
                out_copy(nxt - _K).wait()
                waited.add(nxt - _K)
            in_copy(nxt).start()
    for i in range(n):
        if i not in waited:
            out_copy(i).wait()


def kernel(x):
    B, T, C = x.shape
    rows = B * T
    x2 = x.reshape(rows, C)
    out = pl.pallas_call(
        _body,
        out_shape=jax.ShapeDtypeStruct((rows, C), x.dtype),
        in_specs=[pl.BlockSpec(memory_space=pl.ANY)],
        out_specs=pl.BlockSpec(memory_space=pl.ANY),
        scratch_shapes=[
            pltpu.VMEM((_K, _CHUNK, C), x.dtype),
            pltpu.SemaphoreType.DMA((_K,)),
            pltpu.SemaphoreType.DMA((_K,)),
        ],
    )(x2)
    return out.reshape(B, T, C)
